# Initial kernel scaffold; baseline (speedup 1.0000x reference)
import jax
import jax.numpy as jnp
from jax.experimental import pallas as pl


def kernel(x, edge_index, W1, b1, W2, b2):
    raise NotImplementedError("write your pallas kernel here")


# speedup vs baseline: 21.7208x; 21.7208x over previous
import functools

import jax
import jax.numpy as jnp
from jax import lax
from jax.experimental import pallas as pl
from jax.experimental.pallas import tpu as pltpu
from jax.experimental.pallas import tpu_sc as plsc

N = 10000
E = 320000
D = 128
H = 64
NC = 2
NS = 16
NP = 10112
SLAB = NP // NS
RP = 2560
ROWS_PER_TILE = RP // NS
WIN = 4
NWIN = ROWS_PER_TILE // WIN
DEG_ROWS_PER_TILE = (RP // NC) // NS
NBLK = 1000
GRID = N // NBLK

_mesh = plsc.VectorSubcoreMesh(
    core_axis_name="c", subcore_axis_name="s", num_cores=NC, num_subcores=NS)
_sc_params = pltpu.CompilerParams(use_tc_tiling_on_sc=False)


def _deg_body(dst_h, zeros_h, ones_h, deg_out, deg_sh, idx_v, ones_v, zbuf_v,
              sem):
    c = lax.axis_index("c")
    s = lax.axis_index("s")
    pltpu.sync_copy(zeros_h.at[pl.ds(s * SLAB, SLAB)], zbuf_v)
    pltpu.sync_copy(zbuf_v, deg_sh.at[pl.ds(s * SLAB, SLAB)])
    pltpu.sync_copy(ones_h, ones_v)
    base = c * (RP // NC) + s * DEG_ROWS_PER_TILE
    pltpu.sync_copy(dst_h.at[pl.ds(base, DEG_ROWS_PER_TILE)], idx_v)
    plsc.subcore_barrier()

    def body(w, carry):
        descs = [
            pltpu.async_copy(ones_v, deg_sh.at[idx_v.at[w * 8 + j]], sem,
                             add=True)
            for j in range(8)
        ]
        for d in descs:
            d.wait()
        return carry

    lax.fori_loop(0, DEG_ROWS_PER_TILE // 8, body, 0)
    plsc.subcore_barrier()
    pltpu.sync_copy(deg_sh.at[pl.ds(s * SLAB, SLAB)], zbuf_v)
    pltpu.sync_copy(zbuf_v, deg_out.at[pl.ds(c * NP + s * SLAB, SLAB)])


def _layer_body(g_h, e_h, s_out, table_sh, accum_sh, idx_s, idx_d, rows_v,
                sem_g, sem_s):
    c = lax.axis_index("c")
    s = lax.axis_index("s")
    for k in range(2):
        r0 = s * SLAB + k * (SLAB // 2)
        buf = rows_v.at[pl.ds(0, SLAB // 2)]
        pltpu.sync_copy(g_h.at[c, pl.ds(r0, SLAB // 2)], buf)
        pltpu.sync_copy(buf, table_sh.at[pl.ds(r0, SLAB // 2)])
        pltpu.sync_copy(buf, accum_sh.at[pl.ds(r0, SLAB // 2)])
    plsc.subcore_barrier()

    def body(w, carry):
        row = s * ROWS_PER_TILE + w * WIN
        pltpu.sync_copy(e_h.at[0, pl.ds(row, WIN)], idx_s)
        pltpu.sync_copy(e_h.at[1, pl.ds(row, WIN)], idx_d)
        gets = [
            pltpu.async_copy(table_sh.at[idx_s.at[j]],
                             rows_v.at[pl.ds(j * 128, 128)], sem_g)
            for j in range(WIN)
        ]
        for d in gets:
            d.wait()
        puts = [
            pltpu.async_copy(rows_v.at[pl.ds(j * 128, 128)],
                             accum_sh.at[idx_d.at[j]], sem_s, add=True)
            for j in range(WIN)
        ]
        for d in puts:
            d.wait()
        return carry

    lax.fori_loop(0, NWIN, body, 0)
    plsc.subcore_barrier()
    for k in range(2):
        r0 = s * SLAB + k * (SLAB // 2)
        buf = rows_v.at[pl.ds(0, SLAB // 2)]
        pltpu.sync_copy(accum_sh.at[pl.ds(r0, SLAB // 2)], buf)
        pltpu.sync_copy(buf, s_out.at[c, pl.ds(r0, SLAB // 2)])


_deg_kernel = pl.kernel(
    _deg_body,
    out_type=jax.ShapeDtypeStruct((NC * NP,), jnp.float32),
    mesh=_mesh,
    scratch_types=[
        pltpu.VMEM_SHARED((NP,), jnp.float32),
        pltpu.VMEM((DEG_ROWS_PER_TILE, 128), jnp.int32),
        pltpu.VMEM((128,), jnp.float32),
        pltpu.VMEM((SLAB,), jnp.float32),
        pltpu.SemaphoreType.DMA,
    ],
    compiler_params=_sc_params,
)

_layer_kernel = pl.kernel(
    _layer_body,
    out_type=jax.ShapeDtypeStruct((NC, NP, H), jnp.float32),
    mesh=_mesh,
    scratch_types=[
        pltpu.VMEM_SHARED((NP, H), jnp.float32),
        pltpu.VMEM_SHARED((NP, H), jnp.float32),
        pltpu.VMEM((WIN, 128), jnp.int32),
        pltpu.VMEM((WIN, 128), jnp.int32),
        pltpu.VMEM((WIN * 128, H), jnp.float32),
        pltpu.SemaphoreType.DMA,
        pltpu.SemaphoreType.DMA,
    ],
    compiler_params=_sc_params,
)


def _prep_body(x_ref, w_ref, deg_ref, g_ref, dinv_ref):
    deg = deg_ref[0] + deg_ref[1] + 1.0
    dinv = lax.rsqrt(deg)
    xb = x_ref[...]
    for c in range(NC):
        h = jnp.dot(xb, w_ref[c], preferred_element_type=jnp.float32)
        g_ref[c] = h * dinv
    dinv_ref[...] = dinv


def _mid_body(s_ref, dinv_ref, b_ref, w_ref, g_ref):
    dinv = dinv_ref[...]
    a0 = jnp.maximum(dinv * s_ref[0] + b_ref[0], 0.0)
    a1 = jnp.maximum(dinv * s_ref[1] + b_ref[1], 0.0)
    for o in range(NC):
        h = (jnp.dot(a0, w_ref[0, o], preferred_element_type=jnp.float32)
             + jnp.dot(a1, w_ref[1, o], preferred_element_type=jnp.float32))
        g_ref[o] = h * dinv


def _final_body(s_ref, dinv_ref, b_ref, out_ref):
    dinv = dinv_ref[...]
    o0 = jnp.maximum(dinv * s_ref[0] + b_ref[0], 0.0)
    o1 = jnp.maximum(dinv * s_ref[1] + b_ref[1], 0.0)
    out_ref[...] = jnp.concatenate([o0, o1], axis=1)


_prep_call = pl.pallas_call(
    _prep_body,
    grid=(GRID,),
    in_specs=[
        pl.BlockSpec((NBLK, D), lambda i: (i, 0)),
        pl.BlockSpec((NC, D, H), lambda i: (0, 0, 0)),
        pl.BlockSpec((NC, NBLK, 1), lambda i: (0, i, 0)),
    ],
    out_specs=(
        pl.BlockSpec((NC, NBLK, H), lambda i: (0, i, 0)),
        pl.BlockSpec((NBLK, 1), lambda i: (i, 0)),
    ),
    out_shape=(
        jax.ShapeDtypeStruct((NC, NP, H), jnp.float32),
        jax.ShapeDtypeStruct((NP, 1), jnp.float32),
    ),
)

_mid_call = pl.pallas_call(
    _mid_body,
    grid=(GRID,),
    in_specs=[
        pl.BlockSpec((NC, NBLK, H), lambda i: (0, i, 0)),
        pl.BlockSpec((NBLK, 1), lambda i: (i, 0)),
        pl.BlockSpec((NC, 1, H), lambda i: (0, 0, 0)),
        pl.BlockSpec((NC, NC, H, H), lambda i: (0, 0, 0, 0)),
    ],
    out_specs=pl.BlockSpec((NC, NBLK, H), lambda i: (0, i, 0)),
    out_shape=jax.ShapeDtypeStruct((NC, NP, H), jnp.float32),
)

_final_call = pl.pallas_call(
    _final_body,
    grid=(GRID,),
    in_specs=[
        pl.BlockSpec((NC, NBLK, H), lambda i: (0, i, 0)),
        pl.BlockSpec((NBLK, 1), lambda i: (i, 0)),
        pl.BlockSpec((NC, 1, H), lambda i: (0, 0, 0)),
    ],
    out_specs=pl.BlockSpec((NBLK, D), lambda i: (i, 0)),
    out_shape=jax.ShapeDtypeStruct((N, D), jnp.float32),
)


def kernel(x, edge_index, W1, b1, W2, b2):
    ei = edge_index.astype(jnp.int32)
    npad = RP * 128 - E
    pad = N + (jnp.arange(npad, dtype=jnp.int32) % (NP - N))
    e_pad = jnp.concatenate(
        [ei, jnp.stack([pad, pad])], axis=1).reshape(2, RP, 128)

    w1h = jnp.stack([W1[:, :H], W1[:, H:]])
    w2q = W2.reshape(NC, H, NC, H).transpose(0, 2, 1, 3)
    b1h = jnp.stack([b1[:H], b1[H:]]).reshape(NC, 1, H)
    b2h = jnp.stack([b2[:H], b2[H:]]).reshape(NC, 1, H)
    zeros_np = jnp.zeros((NP,), jnp.float32)
    ones_128 = jnp.ones((128,), jnp.float32)

    degp = _deg_kernel(e_pad[1], zeros_np, ones_128)
    deg3 = degp.reshape(NC, NP, 1)

    g1, dinv = _prep_call(x, w1h, deg3)
    s1 = _layer_kernel(g1, e_pad)
    g2 = _mid_call(s1, dinv, b1h, w2q)
    s2 = _layer_kernel(g2, e_pad)
    return _final_call(s2, dinv, b2h)


# speedup vs baseline: 24.9253x; 1.1475x over previous
import functools

import jax
import jax.numpy as jnp
from jax import lax
from jax.experimental import pallas as pl
from jax.experimental.pallas import tpu as pltpu
from jax.experimental.pallas import tpu_sc as plsc

N = 10000
E = 320000
D = 128
H = 64
NC = 2
NS = 16
NP = 10112
SLAB = NP // NS
RP = 2560
ROWS_PER_TILE = RP // NS
WIN = 4
NWIN = ROWS_PER_TILE // WIN
DEG_ROWS_PER_TILE = (RP // NC) // NS
NBLK = 1000
GRID = N // NBLK

_mesh = plsc.VectorSubcoreMesh(
    core_axis_name="c", subcore_axis_name="s", num_cores=NC, num_subcores=NS)
_sc_params = pltpu.CompilerParams(use_tc_tiling_on_sc=False)


def _deg_body(dst_h, zeros_h, ones_h, deg_out, deg_sh, idx_v, ones_v, zbuf_v,
              sem):
    c = lax.axis_index("c")
    s = lax.axis_index("s")
    pltpu.sync_copy(zeros_h.at[pl.ds(s * SLAB, SLAB)], zbuf_v)
    pltpu.sync_copy(zbuf_v, deg_sh.at[pl.ds(s * SLAB, SLAB)])
    pltpu.sync_copy(ones_h, ones_v)
    base = c * (RP // NC) + s * DEG_ROWS_PER_TILE
    pltpu.sync_copy(dst_h.at[pl.ds(base, DEG_ROWS_PER_TILE)], idx_v)
    plsc.subcore_barrier()

    def body(w, carry):
        descs = [
            pltpu.async_copy(ones_v, deg_sh.at[idx_v.at[w * 8 + j]], sem,
                             add=True)
            for j in range(8)
        ]
        for d in descs:
            d.wait()
        return carry

    lax.fori_loop(0, DEG_ROWS_PER_TILE // 8, body, 0)
    plsc.subcore_barrier()
    pltpu.sync_copy(deg_sh.at[pl.ds(s * SLAB, SLAB)], zbuf_v)
    pltpu.sync_copy(zbuf_v, deg_out.at[pl.ds(c * NP + s * SLAB, SLAB)])


def _layer_body(g_h, e_h, s_out, accum_sh, idx_as, idx_ad, idx_bs, idx_bd,
                buf_a, buf_b, sem_ga, sem_gb, sem_sa, sem_sb):
    c = lax.axis_index("c")
    s = lax.axis_index("s")
    gc = g_h.at[c]
    for k in range(2):
        r0 = s * SLAB + k * (SLAB // 2)
        buf = buf_a.at[pl.ds(0, SLAB // 2)]
        pltpu.sync_copy(gc.at[pl.ds(r0, SLAB // 2)], buf)
        pltpu.sync_copy(buf, accum_sh.at[pl.ds(r0, SLAB // 2)])
    plsc.subcore_barrier()

    def fire_gathers(idx_ref, buf_ref, sem, row):
        pltpu.sync_copy(e_h.at[0, pl.ds(row, WIN)], idx_ref)
        for j in range(WIN):
            pltpu.async_copy(gc.at[idx_ref.at[j]],
                             buf_ref.at[pl.ds(j * 128, 128)], sem)

    def fire_scatters(idx_ref, buf_ref, sem, row):
        pltpu.sync_copy(e_h.at[1, pl.ds(row, WIN)], idx_ref)
        for j in range(WIN):
            pltpu.async_copy(buf_ref.at[pl.ds(j * 128, 128)],
                             accum_sh.at[idx_ref.at[j]], sem, add=True)

    def wait_window(sem, buf_ref):
        pltpu.make_async_copy(gc.at[pl.ds(0, WIN * 128)], buf_ref, sem).wait()

    base = s * ROWS_PER_TILE
    fire_gathers(idx_as, buf_a, sem_ga, base)

    def body(i, carry):
        w0 = base + i * (2 * WIN)
        w1 = w0 + WIN
        wait_window(sem_ga, buf_a)
        fire_scatters(idx_ad, buf_a, sem_sa, w0)

        @pl.when(i > 0)
        def _():
            wait_window(sem_sb, buf_b)
        fire_gathers(idx_bs, buf_b, sem_gb, w1)
        wait_window(sem_gb, buf_b)
        fire_scatters(idx_bd, buf_b, sem_sb, w1)

        @pl.when(i < NWIN // 2 - 1)
        def _():
            wait_window(sem_sa, buf_a)
            fire_gathers(idx_as, buf_a, sem_ga, w0 + 2 * WIN)
        return carry

    lax.fori_loop(0, NWIN // 2, body, 0)
    wait_window(sem_sa, buf_a)
    wait_window(sem_sb, buf_b)
    plsc.subcore_barrier()
    for k in range(2):
        r0 = s * SLAB + k * (SLAB // 2)
        buf = buf_a.at[pl.ds(0, SLAB // 2)]
        pltpu.sync_copy(accum_sh.at[pl.ds(r0, SLAB // 2)], buf)
        pltpu.sync_copy(buf, s_out.at[c, pl.ds(r0, SLAB // 2)])


_deg_kernel = pl.kernel(
    _deg_body,
    out_type=jax.ShapeDtypeStruct((NC * NP,), jnp.float32),
    mesh=_mesh,
    scratch_types=[
        pltpu.VMEM_SHARED((NP,), jnp.float32),
        pltpu.VMEM((DEG_ROWS_PER_TILE, 128), jnp.int32),
        pltpu.VMEM((128,), jnp.float32),
        pltpu.VMEM((SLAB,), jnp.float32),
        pltpu.SemaphoreType.DMA,
    ],
    compiler_params=_sc_params,
)

_layer_kernel = pl.kernel(
    _layer_body,
    out_type=jax.ShapeDtypeStruct((NC, NP, H), jnp.float32),
    mesh=_mesh,
    scratch_types=[
        pltpu.VMEM_SHARED((NP, H), jnp.float32),
        pltpu.VMEM((WIN, 128), jnp.int32),
        pltpu.VMEM((WIN, 128), jnp.int32),
        pltpu.VMEM((WIN, 128), jnp.int32),
        pltpu.VMEM((WIN, 128), jnp.int32),
        pltpu.VMEM((WIN * 128, H), jnp.float32),
        pltpu.VMEM((WIN * 128, H), jnp.float32),
        pltpu.SemaphoreType.DMA,
        pltpu.SemaphoreType.DMA,
        pltpu.SemaphoreType.DMA,
        pltpu.SemaphoreType.DMA,
    ],
    compiler_params=_sc_params,
)


def _prep_body(x_ref, w_ref, deg_ref, g_ref, dinv_ref):
    deg = deg_ref[0] + deg_ref[1] + 1.0
    dinv = lax.rsqrt(deg)
    xb = x_ref[...]
    for c in range(NC):
        h = jnp.dot(xb, w_ref[c], preferred_element_type=jnp.float32)
        g_ref[c] = h * dinv
    dinv_ref[...] = dinv


def _mid_body(s_ref, dinv_ref, b_ref, w_ref, g_ref):
    dinv = dinv_ref[...]
    a0 = jnp.maximum(dinv * s_ref[0] + b_ref[0], 0.0)
    a1 = jnp.maximum(dinv * s_ref[1] + b_ref[1], 0.0)
    for o in range(NC):
        h = (jnp.dot(a0, w_ref[0, o], preferred_element_type=jnp.float32)
             + jnp.dot(a1, w_ref[1, o], preferred_element_type=jnp.float32))
        g_ref[o] = h * dinv


def _final_body(s_ref, dinv_ref, b_ref, out_ref):
    dinv = dinv_ref[...]
    o0 = jnp.maximum(dinv * s_ref[0] + b_ref[0], 0.0)
    o1 = jnp.maximum(dinv * s_ref[1] + b_ref[1], 0.0)
    out_ref[...] = jnp.concatenate([o0, o1], axis=1)


_prep_call = pl.pallas_call(
    _prep_body,
    grid=(GRID,),
    in_specs=[
        pl.BlockSpec((NBLK, D), lambda i: (i, 0)),
        pl.BlockSpec((NC, D, H), lambda i: (0, 0, 0)),
        pl.BlockSpec((NC, NBLK, 1), lambda i: (0, i, 0)),
    ],
    out_specs=(
        pl.BlockSpec((NC, NBLK, H), lambda i: (0, i, 0)),
        pl.BlockSpec((NBLK, 1), lambda i: (i, 0)),
    ),
    out_shape=(
        jax.ShapeDtypeStruct((NC, NP, H), jnp.float32),
        jax.ShapeDtypeStruct((NP, 1), jnp.float32),
    ),
)

_mid_call = pl.pallas_call(
    _mid_body,
    grid=(GRID,),
    in_specs=[
        pl.BlockSpec((NC, NBLK, H), lambda i: (0, i, 0)),
        pl.BlockSpec((NBLK, 1), lambda i: (i, 0)),
        pl.BlockSpec((NC, 1, H), lambda i: (0, 0, 0)),
        pl.BlockSpec((NC, NC, H, H), lambda i: (0, 0, 0, 0)),
    ],
    out_specs=pl.BlockSpec((NC, NBLK, H), lambda i: (0, i, 0)),
    out_shape=jax.ShapeDtypeStruct((NC, NP, H), jnp.float32),
)

_final_call = pl.pallas_call(
    _final_body,
    grid=(GRID,),
    in_specs=[
        pl.BlockSpec((NC, NBLK, H), lambda i: (0, i, 0)),
        pl.BlockSpec((NBLK, 1), lambda i: (i, 0)),
        pl.BlockSpec((NC, 1, H), lambda i: (0, 0, 0)),
    ],
    out_specs=pl.BlockSpec((NBLK, D), lambda i: (i, 0)),
    out_shape=jax.ShapeDtypeStruct((N, D), jnp.float32),
)


def kernel(x, edge_index, W1, b1, W2, b2):
    ei = edge_index.astype(jnp.int32)
    npad = RP * 128 - E
    pad = N + (jnp.arange(npad, dtype=jnp.int32) % (NP - N))
    e_pad = jnp.concatenate(
        [ei, jnp.stack([pad, pad])], axis=1).reshape(2, RP, 128)

    w1h = jnp.stack([W1[:, :H], W1[:, H:]])
    w2q = W2.reshape(NC, H, NC, H).transpose(0, 2, 1, 3)
    b1h = jnp.stack([b1[:H], b1[H:]]).reshape(NC, 1, H)
    b2h = jnp.stack([b2[:H], b2[H:]]).reshape(NC, 1, H)
    zeros_np = jnp.zeros((NP,), jnp.float32)
    ones_128 = jnp.ones((128,), jnp.float32)

    degp = _deg_kernel(e_pad[1], zeros_np, ones_128)
    deg3 = degp.reshape(NC, NP, 1)

    g1, dinv = _prep_call(x, w1h, deg3)
    s1 = _layer_kernel(g1, e_pad)
    g2 = _mid_call(s1, dinv, b1h, w2q)
    s2 = _layer_kernel(g2, e_pad)
    return _final_call(s2, dinv, b2h)


# speedup vs baseline: 29.5026x; 1.1836x over previous
import jax
import jax.numpy as jnp
from jax import lax
from jax.experimental import pallas as pl
from jax.experimental.pallas import tpu as pltpu
from jax.experimental.pallas import tpu_sc as plsc

N = 10000
E = 320000
D = 128
H = 64
NC = 2
NS = 16
NP = 10240
SLAB = NP // NS
RP = 2560
ROWS_PER_TILE = RP // NS
WIN = 5
NWIN = ROWS_PER_TILE // WIN
NBLK = 1024
GRID = NP // NBLK

_mesh = plsc.VectorSubcoreMesh(
    core_axis_name="c", subcore_axis_name="s", num_cores=NC, num_subcores=NS)
_sc_params = pltpu.CompilerParams(use_tc_tiling_on_sc=False)


def _deg_body(dst_h, zeros_h, ones_h, deg_out, deg_sh, idx_v, ones_v, zbuf_v,
              bcast_v, sem):
    c = lax.axis_index("c")
    s = lax.axis_index("s")
    pltpu.sync_copy(zeros_h.at[pl.ds(s * SLAB, SLAB)], zbuf_v)
    pltpu.sync_copy(zbuf_v, deg_sh.at[pl.ds(s * SLAB, SLAB)])
    pltpu.sync_copy(ones_h, ones_v)
    pltpu.sync_copy(dst_h.at[pl.ds(s * ROWS_PER_TILE, ROWS_PER_TILE)], idx_v)
    plsc.subcore_barrier()

    def body(w, carry):
        descs = [
            pltpu.async_copy(ones_v, deg_sh.at[idx_v.at[w * 8 + j]], sem,
                             add=True)
            for j in range(8)
        ]
        for d in descs:
            d.wait()
        return carry

    lax.fori_loop(0, ROWS_PER_TILE // 8, body, 0)
    plsc.subcore_barrier()
    npc = NP // NC // NS
    n0 = c * (NP // NC) + s * npc
    pltpu.sync_copy(deg_sh.at[pl.ds(n0, npc)], zbuf_v.at[pl.ds(0, npc)])

    def bbody(g, carry):
        vals = zbuf_v[pl.ds(g * 16, 16)]
        for i in range(16):
            vec = jnp.broadcast_to(vals[i], (16,))
            for k in range(8):
                bcast_v[g * 16 + i, pl.ds(k * 16, 16)] = vec
        return carry

    lax.fori_loop(0, npc // 16, bbody, 0)
    pltpu.sync_copy(bcast_v, deg_out.at[pl.ds(n0, npc)])


def _layer_body(g2_h, e_h, s_out, accum_sh, idx_as, idx_ad, idx_bs,
                idx_bd, buf_a, buf_b, sem_ga, sem_gb, sem_sa, sem_sb):
    c = lax.axis_index("c")
    s = lax.axis_index("s")

    def build_slab_idx(idx_ref):
        iota2 = lax.iota(jnp.int32, 16) * 2
        for j in range(WIN):
            for k in range(8):
                base = 2 * (s * SLAB + j * 128 + k * 16) + c
                idx_ref[j, pl.ds(k * 16, 16)] = iota2 + base

    build_slab_idx(idx_as)
    for j in range(WIN):
        pltpu.async_copy(g2_h.at[idx_as.at[j]],
                         buf_a.at[pl.ds(j * 128, 128)], sem_ga)
    pltpu.make_async_copy(g2_h.at[pl.ds(0, WIN * 128)], buf_a, sem_ga).wait()
    pltpu.sync_copy(buf_a, accum_sh.at[pl.ds(s * SLAB, SLAB)])
    plsc.subcore_barrier()

    def fire_gathers(idx_ref, buf_ref, sem, row):
        pltpu.sync_copy(e_h.at[0, pl.ds(row, WIN)], idx_ref)
        for j in range(WIN):
            for k in range(8):
                v = idx_ref[j, pl.ds(k * 16, 16)]
                idx_ref[j, pl.ds(k * 16, 16)] = v * 2 + c
        for j in range(WIN):
            pltpu.async_copy(g2_h.at[idx_ref.at[j]],
                             buf_ref.at[pl.ds(j * 128, 128)], sem)

    def fire_scatters(idx_ref, buf_ref, sem, row):
        pltpu.sync_copy(e_h.at[1, pl.ds(row, WIN)], idx_ref)
        for j in range(WIN):
            pltpu.async_copy(buf_ref.at[pl.ds(j * 128, 128)],
                             accum_sh.at[idx_ref.at[j]], sem, add=True)

    def wait_window(sem, buf_ref):
        pltpu.make_async_copy(g2_h.at[pl.ds(0, WIN * 128)], buf_ref,
                              sem).wait()

    base = s * ROWS_PER_TILE
    fire_gathers(idx_as, buf_a, sem_ga, base)

    def body(i, carry):
        w0 = base + i * (2 * WIN)
        w1 = w0 + WIN
        wait_window(sem_ga, buf_a)
        fire_scatters(idx_ad, buf_a, sem_sa, w0)

        @pl.when(i > 0)
        def _():
            wait_window(sem_sb, buf_b)
        fire_gathers(idx_bs, buf_b, sem_gb, w1)
        wait_window(sem_gb, buf_b)
        fire_scatters(idx_bd, buf_b, sem_sb, w1)

        @pl.when(i < NWIN // 2 - 1)
        def _():
            wait_window(sem_sa, buf_a)
            fire_gathers(idx_as, buf_a, sem_ga, w0 + 2 * WIN)
        return carry

    lax.fori_loop(0, NWIN // 2, body, 0)
    wait_window(sem_sa, buf_a)
    wait_window(sem_sb, buf_b)
    plsc.subcore_barrier()
    pltpu.sync_copy(accum_sh.at[pl.ds(s * SLAB, SLAB)], buf_a)
    build_slab_idx(idx_as)
    for j in range(WIN):
        pltpu.async_copy(buf_a.at[pl.ds(j * 128, 128)],
                         s_out.at[idx_as.at[j]], sem_sa)
    pltpu.make_async_copy(g2_h.at[pl.ds(0, WIN * 128)], buf_a, sem_sa).wait()


_deg_kernel = pl.kernel(
    _deg_body,
    out_type=jax.ShapeDtypeStruct((NP, D), jnp.float32),
    mesh=_mesh,
    scratch_types=[
        pltpu.VMEM_SHARED((NP,), jnp.float32),
        pltpu.VMEM((ROWS_PER_TILE, 128), jnp.int32),
        pltpu.VMEM((128,), jnp.float32),
        pltpu.VMEM((SLAB,), jnp.float32),
        pltpu.VMEM((NP // NC // NS, D), jnp.float32),
        pltpu.SemaphoreType.DMA,
    ],
    compiler_params=_sc_params,
)

_layer_kernel = pl.kernel(
    _layer_body,
    out_type=jax.ShapeDtypeStruct((2 * NP, H), jnp.float32),
    mesh=_mesh,
    scratch_types=[
        pltpu.VMEM_SHARED((NP, H), jnp.float32),
        pltpu.VMEM((WIN, 128), jnp.int32),
        pltpu.VMEM((WIN, 128), jnp.int32),
        pltpu.VMEM((WIN, 128), jnp.int32),
        pltpu.VMEM((WIN, 128), jnp.int32),
        pltpu.VMEM((WIN * 128, H), jnp.float32),
        pltpu.VMEM((WIN * 128, H), jnp.float32),
        pltpu.SemaphoreType.DMA,
        pltpu.SemaphoreType.DMA,
        pltpu.SemaphoreType.DMA,
        pltpu.SemaphoreType.DMA,
    ],
    compiler_params=_sc_params,
)


def _prep_body(x_ref, w_ref, deg_ref, g_ref, dinv_ref):
    dinv = lax.rsqrt(deg_ref[...] + 1.0)
    h = jnp.dot(x_ref[...], w_ref[...], preferred_element_type=jnp.float32)
    g_ref[...] = h * dinv
    dinv_ref[...] = dinv


def _mid_body(s_ref, dinv_ref, b_ref, w_ref, g_ref):
    dinv = dinv_ref[...]
    a = jnp.maximum(dinv * s_ref[...] + b_ref[...], 0.0)
    h = jnp.dot(a, w_ref[...], preferred_element_type=jnp.float32)
    g_ref[...] = h * dinv


def _final_body(s_ref, dinv_ref, b_ref, out_ref):
    out_ref[...] = jnp.maximum(dinv_ref[...] * s_ref[...] + b_ref[...], 0.0)


_prep_call = pl.pallas_call(
    _prep_body,
    grid=(GRID,),
    in_specs=[
        pl.BlockSpec((NBLK, D), lambda i: (i, 0)),
        pl.BlockSpec((D, D), lambda i: (0, 0)),
        pl.BlockSpec((NBLK, D), lambda i: (i, 0)),
    ],
    out_specs=(
        pl.BlockSpec((NBLK, D), lambda i: (i, 0)),
        pl.BlockSpec((NBLK, D), lambda i: (i, 0)),
    ),
    out_shape=(
        jax.ShapeDtypeStruct((NP, D), jnp.float32),
        jax.ShapeDtypeStruct((NP, D), jnp.float32),
    ),
)

_mid_call = pl.pallas_call(
    _mid_body,
    grid=(GRID,),
    in_specs=[
        pl.BlockSpec((NBLK, D), lambda i: (i, 0)),
        pl.BlockSpec((NBLK, D), lambda i: (i, 0)),
        pl.BlockSpec((1, D), lambda i: (0, 0)),
        pl.BlockSpec((D, D), lambda i: (0, 0)),
    ],
    out_specs=pl.BlockSpec((NBLK, D), lambda i: (i, 0)),
    out_shape=jax.ShapeDtypeStruct((NP, D), jnp.float32),
)

_final_call = pl.pallas_call(
    _final_body,
    grid=(GRID,),
    in_specs=[
        pl.BlockSpec((NBLK, D), lambda i: (i, 0)),
        pl.BlockSpec((NBLK, D), lambda i: (i, 0)),
        pl.BlockSpec((1, D), lambda i: (0, 0)),
    ],
    out_specs=pl.BlockSpec((NBLK, D), lambda i: (i, 0)),
    out_shape=jax.ShapeDtypeStruct((N, D), jnp.float32),
)


def kernel(x, edge_index, W1, b1, W2, b2):
    ei = edge_index.astype(jnp.int32)
    npad = RP * 128 - E
    pad = N + (jnp.arange(npad, dtype=jnp.int32) % (NP - N))
    e_pad = jnp.concatenate(
        [ei, jnp.stack([pad, pad])], axis=1).reshape(2, RP, 128)

    b1r = b1.reshape(1, D)
    b2r = b2.reshape(1, D)
    zeros_np = jnp.zeros((NP,), jnp.float32)
    ones_128 = jnp.ones((128,), jnp.float32)

    deg_b = _deg_kernel(e_pad[1], zeros_np, ones_128)
    g1, dinv_b = _prep_call(x, W1, deg_b)
    s1 = _layer_kernel(g1.reshape(2 * NP, H), e_pad)
    g2 = _mid_call(s1.reshape(NP, D), dinv_b, b1r, W2)
    s2 = _layer_kernel(g2.reshape(2 * NP, H), e_pad)
    return _final_call(s2.reshape(NP, D), dinv_b, b2r)


# speedup vs baseline: 33.9340x; 1.1502x over previous
import jax
import jax.numpy as jnp
from jax import lax
from jax.experimental import pallas as pl
from jax.experimental.pallas import tpu as pltpu
from jax.experimental.pallas import tpu_sc as plsc

N = 10000
E = 320000
D = 128
H = 64
NC = 2
NS = 16
NP = 10240
SLAB = NP // NS
RP = 2560
ROWS_PER_TILE = RP // NS
WIN = 2
NWIN = ROWS_PER_TILE // WIN
NBUF = 4
NBLK = 1024
GRID = NP // NBLK

_mesh = plsc.VectorSubcoreMesh(
    core_axis_name="c", subcore_axis_name="s", num_cores=NC, num_subcores=NS)
_sc_params = pltpu.CompilerParams(use_tc_tiling_on_sc=False)


def _deg_body(dst_h, zeros_h, ones_h, deg_out, deg_sh, idx_v, ones_v, zbuf_v,
              bcast_v, sem):
    c = lax.axis_index("c")
    s = lax.axis_index("s")
    pltpu.sync_copy(zeros_h.at[pl.ds(s * SLAB, SLAB)], zbuf_v)
    pltpu.sync_copy(zbuf_v, deg_sh.at[pl.ds(s * SLAB, SLAB)])
    pltpu.sync_copy(ones_h, ones_v)
    pltpu.sync_copy(dst_h.at[pl.ds(s * ROWS_PER_TILE, ROWS_PER_TILE)], idx_v)
    plsc.subcore_barrier()

    def body(w, carry):
        descs = [
            pltpu.async_copy(ones_v, deg_sh.at[idx_v.at[w * 8 + j]], sem,
                             add=True)
            for j in range(8)
        ]
        for d in descs:
            d.wait()
        return carry

    lax.fori_loop(0, ROWS_PER_TILE // 8, body, 0)
    plsc.subcore_barrier()
    npc = NP // NC // NS
    n0 = c * (NP // NC) + s * npc
    pltpu.sync_copy(deg_sh.at[pl.ds(n0, npc)], zbuf_v.at[pl.ds(0, npc)])

    def bbody(g, carry):
        vals = zbuf_v[pl.ds(g * 16, 16)]
        for i in range(16):
            vec = jnp.broadcast_to(vals[i], (16,))
            for k in range(8):
                bcast_v[g * 16 + i, pl.ds(k * 16, 16)] = vec
        return carry

    lax.fori_loop(0, npc // 16, bbody, 0)
    pltpu.sync_copy(bcast_v, deg_out.at[pl.ds(n0, npc)])


def _layer_body(g2_h, e_h, s_out, accum_sh, *scr):
    idx_s = scr[0:NBUF]
    idx_d = scr[NBUF:2 * NBUF]
    bufs = scr[2 * NBUF:3 * NBUF]
    sem_g = scr[3 * NBUF:4 * NBUF]
    sem_s = scr[4 * NBUF:5 * NBUF]
    c = lax.axis_index("c")
    s = lax.axis_index("s")
    iota2 = lax.iota(jnp.int32, 16) * 2
    base = s * ROWS_PER_TILE

    def chunk_idx(q, k):
        for kk in range(8):
            b0 = 2 * (s * SLAB + k * 128 + kk * 16) + c
            idx_s[q][0, pl.ds(kk * 16, 16)] = iota2 + b0

    def wait_chunk(sem, q):
        pltpu.make_async_copy(g2_h.at[pl.ds(0, 128)],
                              bufs[q].at[pl.ds(0, 128)], sem).wait()

    for k in range(5):
        q = k % NBUF
        if k >= NBUF:
            wait_chunk(sem_g[0], 0)
            pltpu.sync_copy(bufs[0].at[pl.ds(0, 128)],
                            accum_sh.at[pl.ds(s * SLAB, 128)])
        chunk_idx(q, k)
        pltpu.async_copy(g2_h.at[idx_s[q].at[0]],
                         bufs[q].at[pl.ds(0, 128)], sem_g[q])
    for k in range(1, 5):
        q = k % NBUF
        wait_chunk(sem_g[q], q)
        pltpu.sync_copy(bufs[q].at[pl.ds(0, 128)],
                        accum_sh.at[pl.ds(s * SLAB + k * 128, 128)])
    plsc.subcore_barrier()

    def fire_g(q, w):
        row = base + w * WIN
        pltpu.sync_copy(e_h.at[0, pl.ds(row, WIN)], idx_s[q])
        for j in range(WIN):
            for k in range(8):
                v = idx_s[q][j, pl.ds(k * 16, 16)]
                idx_s[q][j, pl.ds(k * 16, 16)] = v * 2 + c
        for j in range(WIN):
            pltpu.async_copy(g2_h.at[idx_s[q].at[j]],
                             bufs[q].at[pl.ds(j * 128, 128)], sem_g[q])

    def fire_s(q, w):
        row = base + w * WIN
        pltpu.sync_copy(e_h.at[1, pl.ds(row, WIN)], idx_d[q])
        for j in range(WIN):
            pltpu.async_copy(bufs[q].at[pl.ds(j * 128, 128)],
                             accum_sh.at[idx_d[q].at[j]], sem_s[q], add=True)

    def wait_win(sem, q):
        pltpu.make_async_copy(g2_h.at[pl.ds(0, WIN * 128)], bufs[q],
                              sem).wait()

    def body(i, carry):
        for q in range(NBUF):
            w = i * NBUF + q
            @pl.when(i > 0)
            def _(q=q):
                wait_win(sem_s[q], q)
            fire_g(q, w)
            qs = (q - 2) % NBUF
            if q >= 2:
                wait_win(sem_g[qs], qs)
                fire_s(qs, w - 2)
            else:
                @pl.when(i > 0)
                def _(qs=qs, w=w):
                    wait_win(sem_g[qs], qs)
                    fire_s(qs, w - 2)
        return carry

    lax.fori_loop(0, NWIN // NBUF, body, 0)
    wait_win(sem_g[2], 2)
    fire_s(2, NWIN - 2)
    wait_win(sem_g[3], 3)
    fire_s(3, NWIN - 1)
    for q in range(NBUF):
        wait_win(sem_s[q], q)
    plsc.subcore_barrier()
    for k in range(5):
        q = k % NBUF
        if k >= NBUF:
            wait_chunk(sem_s[0], 0)
        pltpu.sync_copy(accum_sh.at[pl.ds(s * SLAB + k * 128, 128)],
                        bufs[q].at[pl.ds(0, 128)])
        chunk_idx(q, k)
        pltpu.async_copy(bufs[q].at[pl.ds(0, 128)],
                         s_out.at[idx_s[q].at[0]], sem_s[q])
    for k in range(1, 5):
        wait_chunk(sem_s[k % NBUF], k % NBUF)


_deg_kernel = pl.kernel(
    _deg_body,
    out_type=jax.ShapeDtypeStruct((NP, D), jnp.float32),
    mesh=_mesh,
    scratch_types=[
        pltpu.VMEM_SHARED((NP,), jnp.float32),
        pltpu.VMEM((ROWS_PER_TILE, 128), jnp.int32),
        pltpu.VMEM((128,), jnp.float32),
        pltpu.VMEM((SLAB,), jnp.float32),
        pltpu.VMEM((NP // NC // NS, D), jnp.float32),
        pltpu.SemaphoreType.DMA,
    ],
    compiler_params=_sc_params,
)

_layer_kernel = pl.kernel(
    _layer_body,
    out_type=jax.ShapeDtypeStruct((2 * NP, H), jnp.float32),
    mesh=_mesh,
    scratch_types=(
        [pltpu.VMEM_SHARED((NP, H), jnp.float32)]
        + [pltpu.VMEM((WIN, 128), jnp.int32) for _ in range(2 * NBUF)]
        + [pltpu.VMEM((WIN * 128, H), jnp.float32) for _ in range(NBUF)]
        + [pltpu.SemaphoreType.DMA for _ in range(2 * NBUF)]
    ),
    compiler_params=_sc_params,
)


def _prep_body(x_ref, w_ref, deg_ref, g_ref, dinv_ref):
    dinv = lax.rsqrt(deg_ref[...] + 1.0)
    h = jnp.dot(x_ref[...], w_ref[...], preferred_element_type=jnp.float32)
    g_ref[...] = h * dinv
    dinv_ref[...] = dinv


def _mid_body(s_ref, dinv_ref, b_ref, w_ref, g_ref):
    dinv = dinv_ref[...]
    a = jnp.maximum(dinv * s_ref[...] + b_ref[...], 0.0)
    h = jnp.dot(a, w_ref[...], preferred_element_type=jnp.float32)
    g_ref[...] = h * dinv


def _final_body(s_ref, dinv_ref, b_ref, out_ref):
    out_ref[...] = jnp.maximum(dinv_ref[...] * s_ref[...] + b_ref[...], 0.0)


_prep_call = pl.pallas_call(
    _prep_body,
    grid=(GRID,),
    in_specs=[
        pl.BlockSpec((NBLK, D), lambda i: (i, 0)),
        pl.BlockSpec((D, D), lambda i: (0, 0)),
        pl.BlockSpec((NBLK, D), lambda i: (i, 0)),
    ],
    out_specs=(
        pl.BlockSpec((NBLK, D), lambda i: (i, 0)),
        pl.BlockSpec((NBLK, D), lambda i: (i, 0)),
    ),
    out_shape=(
        jax.ShapeDtypeStruct((NP, D), jnp.float32),
        jax.ShapeDtypeStruct((NP, D), jnp.float32),
    ),
)

_mid_call = pl.pallas_call(
    _mid_body,
    grid=(GRID,),
    in_specs=[
        pl.BlockSpec((NBLK, D), lambda i: (i, 0)),
        pl.BlockSpec((NBLK, D), lambda i: (i, 0)),
        pl.BlockSpec((1, D), lambda i: (0, 0)),
        pl.BlockSpec((D, D), lambda i: (0, 0)),
    ],
    out_specs=pl.BlockSpec((NBLK, D), lambda i: (i, 0)),
    out_shape=jax.ShapeDtypeStruct((NP, D), jnp.float32),
)

_final_call = pl.pallas_call(
    _final_body,
    grid=(GRID,),
    in_specs=[
        pl.BlockSpec((NBLK, D), lambda i: (i, 0)),
        pl.BlockSpec((NBLK, D), lambda i: (i, 0)),
        pl.BlockSpec((1, D), lambda i: (0, 0)),
    ],
    out_specs=pl.BlockSpec((NBLK, D), lambda i: (i, 0)),
    out_shape=jax.ShapeDtypeStruct((N, D), jnp.float32),
)


def kernel(x, edge_index, W1, b1, W2, b2):
    ei = edge_index.astype(jnp.int32)
    npad = RP * 128 - E
    pad = N + (jnp.arange(npad, dtype=jnp.int32) % (NP - N))
    e_pad = jnp.concatenate(
        [ei, jnp.stack([pad, pad])], axis=1).reshape(2, RP, 128)

    b1r = b1.reshape(1, D)
    b2r = b2.reshape(1, D)
    zeros_np = jnp.zeros((NP,), jnp.float32)
    ones_128 = jnp.ones((128,), jnp.float32)

    deg_b = _deg_kernel(e_pad[1], zeros_np, ones_128)
    g1, dinv_b = _prep_call(x, W1, deg_b)
    s1 = _layer_kernel(g1.reshape(2 * NP, H), e_pad)
    g2 = _mid_call(s1.reshape(NP, D), dinv_b, b1r, W2)
    s2 = _layer_kernel(g2.reshape(2 * NP, H), e_pad)
    return _final_call(s2.reshape(NP, D), dinv_b, b2r)


# speedup vs baseline: 37.8790x; 1.1163x over previous
import jax
import jax.numpy as jnp
from jax import lax
from jax.experimental import pallas as pl
from jax.experimental.pallas import tpu as pltpu
from jax.experimental.pallas import tpu_sc as plsc

N = 10000
E = 320000
D = 128
H = 64
NC = 2
NS = 16
NP = 10240
SLAB = NP // NS
RP = 2560
ROWS_PER_TILE = RP // NS
WIN = 2
NWIN = ROWS_PER_TILE // WIN
NBUF = 4
CHUNK_W = 20
NCHUNK = NWIN // CHUNK_W
NBLK = 1024
GRID = NP // NBLK

_mesh = plsc.VectorSubcoreMesh(
    core_axis_name="c", subcore_axis_name="s", num_cores=NC, num_subcores=NS)
_sc_params = pltpu.CompilerParams(use_tc_tiling_on_sc=False)


def _deg_body(dst_h, zeros_h, ones_h, deg_out, deg_sh, idx_v, ones_v, zbuf_v,
              bcast_v, sem):
    c = lax.axis_index("c")
    s = lax.axis_index("s")
    pltpu.sync_copy(zeros_h.at[pl.ds(s * SLAB, SLAB)], zbuf_v)
    pltpu.sync_copy(zbuf_v, deg_sh.at[pl.ds(s * SLAB, SLAB)])
    pltpu.sync_copy(ones_h, ones_v)
    pltpu.sync_copy(dst_h.at[pl.ds(s * ROWS_PER_TILE, ROWS_PER_TILE)], idx_v)
    plsc.subcore_barrier()

    def body(w, carry):
        descs = [
            pltpu.async_copy(ones_v, deg_sh.at[idx_v.at[w * 8 + j]], sem,
                             add=True)
            for j in range(8)
        ]
        for d in descs:
            d.wait()
        return carry

    lax.fori_loop(0, ROWS_PER_TILE // 8, body, 0)
    plsc.subcore_barrier()
    npc = NP // NC // NS
    n0 = c * (NP // NC) + s * npc
    pltpu.sync_copy(deg_sh.at[pl.ds(n0, npc)], zbuf_v.at[pl.ds(0, npc)])

    def bbody(g, carry):
        vals = zbuf_v[pl.ds(g * 16, 16)]
        for i in range(16):
            vec = jnp.broadcast_to(vals[i], (16,))
            for k in range(8):
                bcast_v[g * 16 + i, pl.ds(k * 16, 16)] = vec
        return carry

    lax.fori_loop(0, npc // 16, bbody, 0)
    pltpu.sync_copy(bcast_v, deg_out.at[pl.ds(n0, npc)])


def _layer_body(g2_h, e_h, s_out, accum_sh, *scr):
    bufs = scr[0:NBUF]
    es = scr[NBUF:NBUF + 2]
    ed = scr[NBUF + 2:NBUF + 4]
    stg_idx = scr[NBUF + 4]
    sem_g = scr[NBUF + 5:2 * NBUF + 5]
    sem_s = scr[2 * NBUF + 5:3 * NBUF + 5]
    sem_e = scr[3 * NBUF + 5:3 * NBUF + 7]
    c = lax.axis_index("c")
    s = lax.axis_index("s")
    iota2 = lax.iota(jnp.int32, 16) * 2
    base = s * ROWS_PER_TILE
    crows = CHUNK_W * WIN

    def fire_echunk(ch, p):
        row = base + ch * crows
        pltpu.async_copy(e_h.at[0, pl.ds(row, crows)], es[p], sem_e[p])
        pltpu.async_copy(e_h.at[1, pl.ds(row, crows)], ed[p], sem_e[p])

    def wait_echunk(p):
        for _ in range(2):
            pltpu.make_async_copy(e_h.at[0, pl.ds(0, crows)], es[p],
                                  sem_e[p]).wait()

    fire_echunk(0, 0)

    def chunk_idx(k):
        for kk in range(8):
            b0 = 2 * (s * SLAB + k * 128 + kk * 16) + c
            stg_idx[k, pl.ds(kk * 16, 16)] = iota2 + b0

    def wait_chunk(sem, q):
        pltpu.make_async_copy(g2_h.at[pl.ds(0, 128)],
                              bufs[q].at[pl.ds(0, 128)], sem).wait()

    for k in range(5):
        q = k % NBUF
        if k >= NBUF:
            wait_chunk(sem_g[0], 0)
            pltpu.sync_copy(bufs[0].at[pl.ds(0, 128)],
                            accum_sh.at[pl.ds(s * SLAB, 128)])
        chunk_idx(k)
        pltpu.async_copy(g2_h.at[stg_idx.at[k]],
                         bufs[q].at[pl.ds(0, 128)], sem_g[q])
    for k in range(1, 5):
        q = k % NBUF
        wait_chunk(sem_g[q], q)
        pltpu.sync_copy(bufs[q].at[pl.ds(0, 128)],
                        accum_sh.at[pl.ds(s * SLAB + k * 128, 128)])
    plsc.subcore_barrier()

    def fire_g(q, es_ref, lrow):
        for j in range(WIN):
            pltpu.async_copy(g2_h.at[es_ref.at[lrow + j]],
                             bufs[q].at[pl.ds(j * 128, 128)], sem_g[q])

    def fire_s(q, ed_ref, lrow):
        for j in range(WIN):
            pltpu.async_copy(bufs[q].at[pl.ds(j * 128, 128)],
                             accum_sh.at[ed_ref.at[lrow + j]], sem_s[q],
                             add=True)

    def wait_win(sem, q):
        pltpu.make_async_copy(g2_h.at[pl.ds(0, WIN * 128)], bufs[q],
                              sem).wait()

    def win_body(i, ch, p, first):
        for q in range(NBUF):
            lw = i * NBUF + q
            if not first:
                wait_win(sem_s[q], q)
            fire_g(q, es[p], lw * WIN)
            qs = (q - 2) % NBUF
            if first and q < 2:
                continue
            wait_win(sem_g[qs], qs)
            if isinstance(i, int) and i == 0 and q < 2:
                fire_s(qs, ed[1 - p], (crows - 4 + q * WIN))
            else:
                fire_s(qs, ed[p], (lw - 2) * WIN)

    for ch in range(NCHUNK):
        p = ch % 2
        wait_echunk(p)

        def tbody(r, carry):
            for k in range(8):
                v = es[p][r, pl.ds(k * 16, 16)]
                es[p][r, pl.ds(k * 16, 16)] = v * 2 + c
            return carry

        lax.fori_loop(0, crows, tbody, 0)
        win_body(0, ch, p, first=(ch == 0))
        if ch + 1 < NCHUNK:
            fire_echunk(ch + 1, 1 - p)

        def ibody(i, carry, ch=ch, p=p):
            win_body(i, ch, p, first=False)
            return carry

        lax.fori_loop(1, CHUNK_W // NBUF, ibody, 0)

    pl_last = (NCHUNK - 1) % 2
    wait_win(sem_g[2], 2)
    fire_s(2, ed[pl_last], crows - 4)
    wait_win(sem_g[3], 3)
    fire_s(3, ed[pl_last], crows - 2)
    for q in range(NBUF):
        wait_win(sem_s[q], q)
    plsc.subcore_barrier()
    for k in range(5):
        q = k % NBUF
        if k >= NBUF:
            wait_chunk(sem_s[0], 0)
        pltpu.sync_copy(accum_sh.at[pl.ds(s * SLAB + k * 128, 128)],
                        bufs[q].at[pl.ds(0, 128)])
        chunk_idx(k)
        pltpu.async_copy(bufs[q].at[pl.ds(0, 128)],
                         s_out.at[stg_idx.at[k]], sem_s[q])
    for k in range(1, 5):
        wait_chunk(sem_s[k % NBUF], k % NBUF)


_deg_kernel = pl.kernel(
    _deg_body,
    out_type=jax.ShapeDtypeStruct((NP, D), jnp.float32),
    mesh=_mesh,
    scratch_types=[
        pltpu.VMEM_SHARED((NP,), jnp.float32),
        pltpu.VMEM((ROWS_PER_TILE, 128), jnp.int32),
        pltpu.VMEM((128,), jnp.float32),
        pltpu.VMEM((SLAB,), jnp.float32),
        pltpu.VMEM((NP // NC // NS, D), jnp.float32),
        pltpu.SemaphoreType.DMA,
    ],
    compiler_params=_sc_params,
)

_layer_kernel = pl.kernel(
    _layer_body,
    out_type=jax.ShapeDtypeStruct((2 * NP, H), jnp.float32),
    mesh=_mesh,
    scratch_types=(
        [pltpu.VMEM_SHARED((NP, H), jnp.float32)]
        + [pltpu.VMEM((WIN * 128, H), jnp.float32) for _ in range(NBUF)]
        + [pltpu.VMEM((CHUNK_W * WIN, 128), jnp.int32) for _ in range(4)]
        + [pltpu.VMEM((5, 128), jnp.int32)]
        + [pltpu.SemaphoreType.DMA for _ in range(2 * NBUF + 2)]
    ),
    compiler_params=_sc_params,
)


def _prep_body(x_ref, w_ref, deg_ref, g_ref, dinv_ref):
    dinv = lax.rsqrt(deg_ref[...] + 1.0)
    h = jnp.dot(x_ref[...], w_ref[...], preferred_element_type=jnp.float32)
    g_ref[...] = h * dinv
    dinv_ref[...] = dinv


def _mid_body(s_ref, dinv_ref, b_ref, w_ref, g_ref):
    dinv = dinv_ref[...]
    a = jnp.maximum(dinv * s_ref[...] + b_ref[...], 0.0)
    h = jnp.dot(a, w_ref[...], preferred_element_type=jnp.float32)
    g_ref[...] = h * dinv


def _final_body(s_ref, dinv_ref, b_ref, out_ref):
    out_ref[...] = jnp.maximum(dinv_ref[...] * s_ref[...] + b_ref[...], 0.0)


_prep_call = pl.pallas_call(
    _prep_body,
    grid=(GRID,),
    in_specs=[
        pl.BlockSpec((NBLK, D), lambda i: (i, 0)),
        pl.BlockSpec((D, D), lambda i: (0, 0)),
        pl.BlockSpec((NBLK, D), lambda i: (i, 0)),
    ],
    out_specs=(
        pl.BlockSpec((NBLK, D), lambda i: (i, 0)),
        pl.BlockSpec((NBLK, D), lambda i: (i, 0)),
    ),
    out_shape=(
        jax.ShapeDtypeStruct((NP, D), jnp.float32),
        jax.ShapeDtypeStruct((NP, D), jnp.float32),
    ),
)

_mid_call = pl.pallas_call(
    _mid_body,
    grid=(GRID,),
    in_specs=[
        pl.BlockSpec((NBLK, D), lambda i: (i, 0)),
        pl.BlockSpec((NBLK, D), lambda i: (i, 0)),
        pl.BlockSpec((1, D), lambda i: (0, 0)),
        pl.BlockSpec((D, D), lambda i: (0, 0)),
    ],
    out_specs=pl.BlockSpec((NBLK, D), lambda i: (i, 0)),
    out_shape=jax.ShapeDtypeStruct((NP, D), jnp.float32),
)

_final_call = pl.pallas_call(
    _final_body,
    grid=(GRID,),
    in_specs=[
        pl.BlockSpec((NBLK, D), lambda i: (i, 0)),
        pl.BlockSpec((NBLK, D), lambda i: (i, 0)),
        pl.BlockSpec((1, D), lambda i: (0, 0)),
    ],
    out_specs=pl.BlockSpec((NBLK, D), lambda i: (i, 0)),
    out_shape=jax.ShapeDtypeStruct((N, D), jnp.float32),
)


def kernel(x, edge_index, W1, b1, W2, b2):
    ei = edge_index.astype(jnp.int32)
    npad = RP * 128 - E
    pad = N + (jnp.arange(npad, dtype=jnp.int32) % (NP - N))
    e_pad = jnp.concatenate(
        [ei, jnp.stack([pad, pad])], axis=1).reshape(2, RP, 128)

    b1r = b1.reshape(1, D)
    b2r = b2.reshape(1, D)
    zeros_np = jnp.zeros((NP,), jnp.float32)
    ones_128 = jnp.ones((128,), jnp.float32)

    deg_b = _deg_kernel(e_pad[1], zeros_np, ones_128)
    g1, dinv_b = _prep_call(x, W1, deg_b)
    s1 = _layer_kernel(g1.reshape(2 * NP, H), e_pad)
    g2 = _mid_call(s1.reshape(NP, D), dinv_b, b1r, W2)
    s2 = _layer_kernel(g2.reshape(2 * NP, H), e_pad)
    return _final_call(s2.reshape(NP, D), dinv_b, b2r)


# speedup vs baseline: 37.9838x; 1.0028x over previous
import jax
import jax.numpy as jnp
from jax import lax
from jax.experimental import pallas as pl
from jax.experimental.pallas import tpu as pltpu
from jax.experimental.pallas import tpu_sc as plsc

N = 10000
E = 320000
D = 128
H = 64
NC = 2
NS = 16
NP = 10240
SLAB = NP // NS
RP = 2560
ROWS_PER_TILE = RP // NS
WIN = 2
NWIN = ROWS_PER_TILE // WIN
NBUF = 4
CHUNK_W = 20
NCHUNK = NWIN // CHUNK_W
NBLK = 1024
GRID = NP // NBLK

_mesh = plsc.VectorSubcoreMesh(
    core_axis_name="c", subcore_axis_name="s", num_cores=NC, num_subcores=NS)
_sc_params = pltpu.CompilerParams(use_tc_tiling_on_sc=False)


def _deg_body(dst_h, zeros_h, ones_h, deg_out, deg_sh, idx_v, ones_v, zbuf_v,
              bcast_v, sem):
    c = lax.axis_index("c")
    s = lax.axis_index("s")
    pltpu.sync_copy(zeros_h.at[pl.ds(s * SLAB, SLAB)], zbuf_v)
    pltpu.sync_copy(zbuf_v, deg_sh.at[pl.ds(s * SLAB, SLAB)])
    pltpu.sync_copy(ones_h, ones_v)
    pltpu.sync_copy(dst_h.at[pl.ds(s * ROWS_PER_TILE, ROWS_PER_TILE)], idx_v)
    plsc.subcore_barrier()

    def body(w, carry):
        descs = [
            pltpu.async_copy(ones_v, deg_sh.at[idx_v.at[w * 8 + j]], sem,
                             add=True)
            for j in range(8)
        ]
        for d in descs:
            d.wait()
        return carry

    lax.fori_loop(0, ROWS_PER_TILE // 8, body, 0)
    plsc.subcore_barrier()
    npc = NP // NC // NS
    n0 = c * (NP // NC) + s * npc
    pltpu.sync_copy(deg_sh.at[pl.ds(n0, npc)], zbuf_v.at[pl.ds(0, npc)])

    def bbody(g, carry):
        vals = zbuf_v[pl.ds(g * 16, 16)]
        for i in range(16):
            vec = jnp.broadcast_to(vals[i], (16,))
            for k in range(8):
                bcast_v[g * 16 + i, pl.ds(k * 16, 16)] = vec
        return carry

    lax.fori_loop(0, npc // 16, bbody, 0)
    pltpu.sync_copy(bcast_v, deg_out.at[pl.ds(n0, npc)])


def _layer_body(g2_h, e_h, s_out, accum_sh, *scr):
    bufs = scr[0:NBUF]
    es = scr[NBUF:NBUF + 2]
    ed = scr[NBUF + 2:NBUF + 4]
    stg_idx = scr[NBUF + 4]
    sem_g = scr[NBUF + 5:2 * NBUF + 5]
    sem_s = scr[2 * NBUF + 5:3 * NBUF + 5]
    sem_e = scr[3 * NBUF + 5:3 * NBUF + 7]
    c = lax.axis_index("c")
    s = lax.axis_index("s")
    iota2 = lax.iota(jnp.int32, 16) * 2
    base = s * ROWS_PER_TILE
    crows = CHUNK_W * WIN

    def fire_echunk(ch, p):
        row = base + ch * crows
        pltpu.async_copy(e_h.at[0, pl.ds(row, crows)], es[p], sem_e[p])
        pltpu.async_copy(e_h.at[1, pl.ds(row, crows)], ed[p], sem_e[p])

    def wait_echunk(p):
        for _ in range(2):
            pltpu.make_async_copy(e_h.at[0, pl.ds(0, crows)], es[p],
                                  sem_e[p]).wait()

    fire_echunk(0, 0)

    def chunk_idx(k):
        for kk in range(8):
            b0 = 2 * (s * SLAB + k * 128 + kk * 16) + c
            stg_idx[k, pl.ds(kk * 16, 16)] = iota2 + b0

    def wait_chunk(sem, q):
        pltpu.make_async_copy(g2_h.at[pl.ds(0, 128)],
                              bufs[q].at[pl.ds(0, 128)], sem).wait()

    for k in range(5):
        q = k % NBUF
        if k >= NBUF:
            wait_chunk(sem_g[0], 0)
            pltpu.sync_copy(bufs[0].at[pl.ds(0, 128)],
                            accum_sh.at[pl.ds(s * SLAB, 128)])
        chunk_idx(k)
        pltpu.async_copy(g2_h.at[stg_idx.at[k]],
                         bufs[q].at[pl.ds(0, 128)], sem_g[q])
    for k in range(1, 5):
        q = k % NBUF
        wait_chunk(sem_g[q], q)
        pltpu.sync_copy(bufs[q].at[pl.ds(0, 128)],
                        accum_sh.at[pl.ds(s * SLAB + k * 128, 128)])
    plsc.subcore_barrier()

    def fire_g(q, es_ref, lrow):
        for j in range(WIN):
            pltpu.async_copy(g2_h.at[es_ref.at[lrow + j]],
                             bufs[q].at[pl.ds(j * 128, 128)], sem_g[q])

    def fire_s(q, ed_ref, lrow):
        for j in range(WIN):
            pltpu.async_copy(bufs[q].at[pl.ds(j * 128, 128)],
                             accum_sh.at[ed_ref.at[lrow + j]], sem_s[q],
                             add=True)

    def wait_win(sem, q):
        pltpu.make_async_copy(g2_h.at[pl.ds(0, WIN * 128)], bufs[q],
                              sem).wait()

    def win_body(i, ch, p, first):
        for q in range(NBUF):
            lw = i * NBUF + q
            if not first:
                wait_win(sem_s[q], q)
            fire_g(q, es[p], lw * WIN)
            qs = (q - 2) % NBUF
            if first and q < 2:
                continue
            wait_win(sem_g[qs], qs)
            if isinstance(i, int) and i == 0 and q < 2:
                fire_s(qs, ed[1 - p], (crows - 4 + q * WIN))
            else:
                fire_s(qs, ed[p], (lw - 2) * WIN)

    for ch in range(NCHUNK):
        p = ch % 2
        wait_echunk(p)

        def tbody(r, carry):
            for k in range(8):
                v = es[p][r, pl.ds(k * 16, 16)]
                es[p][r, pl.ds(k * 16, 16)] = v * 2 + c
            return carry

        lax.fori_loop(0, crows, tbody, 0)
        win_body(0, ch, p, first=(ch == 0))
        if ch + 1 < NCHUNK:
            fire_echunk(ch + 1, 1 - p)

        def ibody(i, carry, ch=ch, p=p):
            win_body(i, ch, p, first=False)
            return carry

        lax.fori_loop(1, CHUNK_W // NBUF, ibody, 0)

    pl_last = (NCHUNK - 1) % 2
    wait_win(sem_g[2], 2)
    fire_s(2, ed[pl_last], crows - 4)
    wait_win(sem_g[3], 3)
    fire_s(3, ed[pl_last], crows - 2)
    for q in range(NBUF):
        wait_win(sem_s[q], q)
    plsc.subcore_barrier()
    for k in range(5):
        q = k % NBUF
        if k >= NBUF:
            wait_chunk(sem_s[0], 0)
        pltpu.sync_copy(accum_sh.at[pl.ds(s * SLAB + k * 128, 128)],
                        bufs[q].at[pl.ds(0, 128)])
        chunk_idx(k)
        pltpu.async_copy(bufs[q].at[pl.ds(0, 128)],
                         s_out.at[stg_idx.at[k]], sem_s[q])
    for k in range(1, 5):
        wait_chunk(sem_s[k % NBUF], k % NBUF)


_deg_kernel = pl.kernel(
    _deg_body,
    out_type=jax.ShapeDtypeStruct((NP, D), jnp.float32),
    mesh=_mesh,
    scratch_types=[
        pltpu.VMEM_SHARED((NP,), jnp.float32),
        pltpu.VMEM((ROWS_PER_TILE, 128), jnp.int32),
        pltpu.VMEM((128,), jnp.float32),
        pltpu.VMEM((SLAB,), jnp.float32),
        pltpu.VMEM((NP // NC // NS, D), jnp.float32),
        pltpu.SemaphoreType.DMA,
    ],
    compiler_params=_sc_params,
)

_layer_kernel = pl.kernel(
    _layer_body,
    out_type=jax.ShapeDtypeStruct((2 * NP, H), jnp.float32),
    mesh=_mesh,
    scratch_types=(
        [pltpu.VMEM_SHARED((NP, H), jnp.float32)]
        + [pltpu.VMEM((WIN * 128, H), jnp.float32) for _ in range(NBUF)]
        + [pltpu.VMEM((CHUNK_W * WIN, 128), jnp.int32) for _ in range(4)]
        + [pltpu.VMEM((5, 128), jnp.int32)]
        + [pltpu.SemaphoreType.DMA for _ in range(2 * NBUF + 2)]
    ),
    compiler_params=_sc_params,
)


def _prep_body(x_ref, w_ref, deg_ref, g_ref):
    dinv = lax.rsqrt(deg_ref[...] + 1.0)
    h = jnp.dot(x_ref[...], w_ref[...], preferred_element_type=jnp.float32)
    g_ref[...] = h * dinv


def _mid_body(s_ref, deg_ref, b_ref, w_ref, g_ref):
    dinv = lax.rsqrt(deg_ref[...] + 1.0)
    a = jnp.maximum(dinv * s_ref[...] + b_ref[...], 0.0)
    h = jnp.dot(a, w_ref[...], preferred_element_type=jnp.float32)
    g_ref[...] = h * dinv


def _final_body(s_ref, deg_ref, b_ref, out_ref):
    dinv = lax.rsqrt(deg_ref[...] + 1.0)
    out_ref[...] = jnp.maximum(dinv * s_ref[...] + b_ref[...], 0.0)


_prep_call = pl.pallas_call(
    _prep_body,
    grid=(GRID,),
    in_specs=[
        pl.BlockSpec((NBLK, D), lambda i: (i, 0)),
        pl.BlockSpec((D, D), lambda i: (0, 0)),
        pl.BlockSpec((NBLK, D), lambda i: (i, 0)),
    ],
    out_specs=pl.BlockSpec((NBLK, D), lambda i: (i, 0)),
    out_shape=jax.ShapeDtypeStruct((NP, D), jnp.float32),
)

_mid_call = pl.pallas_call(
    _mid_body,
    grid=(GRID,),
    in_specs=[
        pl.BlockSpec((NBLK, D), lambda i: (i, 0)),
        pl.BlockSpec((NBLK, D), lambda i: (i, 0)),
        pl.BlockSpec((1, D), lambda i: (0, 0)),
        pl.BlockSpec((D, D), lambda i: (0, 0)),
    ],
    out_specs=pl.BlockSpec((NBLK, D), lambda i: (i, 0)),
    out_shape=jax.ShapeDtypeStruct((NP, D), jnp.float32),
)

_final_call = pl.pallas_call(
    _final_body,
    grid=(GRID,),
    in_specs=[
        pl.BlockSpec((NBLK, D), lambda i: (i, 0)),
        pl.BlockSpec((NBLK, D), lambda i: (i, 0)),
        pl.BlockSpec((1, D), lambda i: (0, 0)),
    ],
    out_specs=pl.BlockSpec((NBLK, D), lambda i: (i, 0)),
    out_shape=jax.ShapeDtypeStruct((N, D), jnp.float32),
)


def kernel(x, edge_index, W1, b1, W2, b2):
    ei = edge_index.astype(jnp.int32)
    npad = RP * 128 - E
    pad = N + (jnp.arange(npad, dtype=jnp.int32) % (NP - N))
    e_pad = jnp.concatenate(
        [ei, jnp.stack([pad, pad])], axis=1).reshape(2, RP, 128)

    b1r = b1.reshape(1, D)
    b2r = b2.reshape(1, D)
    zeros_np = jnp.zeros((NP,), jnp.float32)
    ones_128 = jnp.ones((128,), jnp.float32)

    deg_b = _deg_kernel(e_pad[1], zeros_np, ones_128)
    g1 = _prep_call(x, W1, deg_b)
    s1 = _layer_kernel(g1.reshape(2 * NP, H), e_pad)
    g2 = _mid_call(s1.reshape(NP, D), deg_b, b1r, W2)
    s2 = _layer_kernel(g2.reshape(2 * NP, H), e_pad)
    return _final_call(s2.reshape(NP, D), deg_b, b2r)
